# Initial kernel scaffold; baseline (speedup 1.0000x reference)
import jax
import jax.numpy as jnp
from jax.experimental import pallas as pl


def kernel(pc_transformed, nn_distances, nn_indices):
    raise NotImplementedError("write your pallas kernel here")


# speedup vs baseline: 8.8023x; 8.8023x over previous
import dataclasses
import functools

import jax
import jax.numpy as jnp
from jax import lax
from jax.experimental import pallas as pl
from jax.experimental.pallas import tpu as pltpu
from jax.experimental.pallas import tpu_sc as plsc

N = 10000
K = 16
L = 16
NC = 2
NS = 16
NW = NC * NS
NPW = 320
NPAD = NW * NPW


def _arap_tec(pc_hbm, idx_hbm, dist_hbm, out_hbm, pc_v, idx_v, dist_v, acc_v):
    cid = lax.axis_index("c")
    sid = lax.axis_index("s")
    wid = sid * NC + cid
    start = wid * NPW

    pltpu.sync_copy(pc_hbm, pc_v.at[pl.ds(0, 3 * N)])
    pltpu.sync_copy(idx_hbm.at[pl.ds(start, NPW)], idx_v)
    pltpu.sync_copy(dist_hbm.at[pl.ds(start, NPW)], dist_v)

    n_valid = jnp.minimum(NPW, N - start)

    def body(t, acc):
        i = start + t
        idx_row = idx_v[t, :]
        base = idx_row * 3
        gx = plsc.load_gather(pc_v, [base])
        gy = plsc.load_gather(pc_v, [base + 1])
        gz = plsc.load_gather(pc_v, [base + 2])
        c = pc_v[pl.ds(3 * i, L)]
        dx = c[0] - gx
        dy = c[1] - gy
        dz = c[2] - gz
        d2 = dx * dx + dy * dy + dz * dz
        return acc + jnp.abs(d2 - dist_v[t, :])

    acc = lax.fori_loop(0, n_valid, body, jnp.zeros((L,), jnp.float32))
    acc_v[...] = acc
    pltpu.sync_copy(acc_v, out_hbm.at[wid])


@functools.partial(jax.jit, static_argnums=())
def _arap_sc(pc_flat, idx_pad, dist_pad):
    cp = pltpu.CompilerParams()
    if "needs_layout_passes" in pltpu.CompilerParams.__dataclass_fields__:
        cp = dataclasses.replace(cp, needs_layout_passes=False)
    run = pl.kernel(
        _arap_tec,
        out_type=jax.ShapeDtypeStruct((NW, L), jnp.float32),
        compiler_params=cp,
        mesh=plsc.VectorSubcoreMesh(core_axis_name="c", subcore_axis_name="s"),
        scratch_types=[
            pltpu.VMEM((3 * N + L,), jnp.float32),

            pltpu.VMEM((NPW, K), jnp.int32),
            pltpu.VMEM((NPW, K), jnp.float32),
            pltpu.VMEM((L,), jnp.float32),
        ],
    )
    return run(pc_flat, idx_pad, dist_pad)


def kernel(pc_transformed, nn_distances, nn_indices):
    pc_flat = pc_transformed.reshape(-1)
    idx = jnp.pad(nn_indices.astype(jnp.int32), ((0, NPAD - N), (0, 0)))
    dist = jnp.pad(nn_distances, ((0, NPAD - N), (0, 0)))
    partials = _arap_sc(pc_flat, idx, dist)
    return jnp.sum(partials) / (N * K)
